# Initial kernel scaffold; baseline (speedup 1.0000x reference)
import jax
import jax.numpy as jnp
from jax.experimental import pallas as pl


def kernel(query, value, reference_points, spatial_shapes, level_start_index, W_so, b_so, W_aw, b_aw, W_v, b_v, W_o, b_o):
    raise NotImplementedError("write your pallas kernel here")


# speedup vs baseline: 80.3759x; 80.3759x over previous
import functools

import numpy as np

import jax
import jax.numpy as jnp
from jax import lax
from jax.experimental import pallas as pl
from jax.experimental.pallas import tpu as pltpu
from jax.experimental.pallas import tpu_sc as plsc

EMBED = 256
HEADS = 8
LEVELS = 4
POINTS = 8
HEAD_DIM = 32
LP = LEVELS * POINTS
NQ = 10000
NV = 21760
NPAIR = NQ * HEADS
NCORNER = LP * 4
_LVL_W = np.array([128, 64, 32, 16], dtype=np.int32)
_STARTS = np.array([0, 16384, 20480, 21504], dtype=np.int32)

_ch = np.arange(EMBED)
_l_of = (_ch // POINTS) % LEVELS
_WC_I = np.asarray(_LVL_W[_l_of], np.int32)[None]
_WC_F = _WC_I.astype(np.float32)
_START_C = np.asarray(_STARTS[_l_of], np.int32)[None]
_HOFF_C = np.asarray((_ch // LP) * NV, np.int32)[None]
_S_LVL = np.zeros((LEVELS, EMBED), np.float32)
_S_LVL[_l_of, _ch] = 1.0
_PERM = np.concatenate([2 * np.arange(EMBED), 2 * np.arange(EMBED) + 1])

BQ = 1000
BV = 680


def _vproj_body(v_ref, wvT_ref, bv_ref, out_ref):
    res = jnp.dot(v_ref[...], wvT_ref[...], preferred_element_type=jnp.float32, precision=lax.Precision.HIGHEST)
    res = res + bv_ref[...]
    for h in range(HEADS):
        out_ref[h] = res[:, h * HEAD_DIM:(h + 1) * HEAD_DIM]


def _locs_body(q_ref, refx_ref, refy_ref, wsoT_ref, bso_ref, wawT_ref, baw_ref,
               slvl_ref, wcf_ref, wci_ref, base_ref, idx_ref, wgt_ref):
    wc_f = wcf_ref[...]
    wc_i = wci_ref[...]
    base_c = base_ref[...]

    q = q_ref[...]
    so = jnp.dot(q, wsoT_ref[...], preferred_element_type=jnp.float32, precision=lax.Precision.HIGHEST) + bso_ref[...]
    aw = jnp.dot(q, wawT_ref[...], preferred_element_type=jnp.float32, precision=lax.Precision.HIGHEST) + baw_ref[...]
    aw3 = aw.reshape(BQ, HEADS, LP)
    aw3 = aw3 - jnp.max(aw3, axis=-1, keepdims=True)
    e = jnp.exp(aw3)
    aw = (e / jnp.sum(e, axis=-1, keepdims=True)).reshape(BQ, EMBED)

    rx = jnp.dot(refx_ref[...], slvl_ref[...], preferred_element_type=jnp.float32, precision=lax.Precision.HIGHEST)
    ry = jnp.dot(refy_ref[...], slvl_ref[...], preferred_element_type=jnp.float32, precision=lax.Precision.HIGHEST)
    x = rx * wc_f + so[:, :EMBED] - 0.5
    y = ry * wc_f + so[:, EMBED:] - 0.5
    x = jnp.clip(x, -2.0, wc_f + 1.0)
    y = jnp.clip(y, -2.0, wc_f + 1.0)
    x0 = jnp.floor(x)
    y0 = jnp.floor(y)
    fx = x - x0
    fy = y - y0
    ix = x0.astype(jnp.int32)
    iy = y0.astype(jnp.int32)

    idxs = []
    ws = []
    for dy in (0, 1):
        for dx in (0, 1):
            xi = ix + dx
            yi = iy + dy
            valid = (xi >= 0) & (xi < wc_i) & (yi >= 0) & (yi < wc_i)
            wx = fx if dx else (1.0 - fx)
            wy = fy if dy else (1.0 - fy)
            w = wx * wy * aw * valid.astype(jnp.float32)
            xc = jnp.clip(xi, 0, wc_i - 1)
            yc = jnp.clip(yi, 0, wc_i - 1)
            idxs.append(base_c + yc * wc_i + xc)
            ws.append(w)
    for h in range(HEADS):
        sl = slice(h * LP, (h + 1) * LP)
        idx_ref[h] = jnp.concatenate([c[:, sl] for c in idxs], axis=-1)
        wgt_ref[h] = jnp.concatenate([c[:, sl] for c in ws], axis=-1)


def _out_body(sc_ref, q_ref, woT_ref, bo_ref, out_ref):
    sc = jnp.concatenate([sc_ref[h] for h in range(HEADS)], axis=-1)
    res = jnp.dot(sc, woT_ref[...], preferred_element_type=jnp.float32, precision=lax.Precision.HIGHEST)
    out_ref[...] = res + bo_ref[...] + q_ref[...]


NW = 32
CHUNK = 8
NCHUNK_HI = 313
NCHUNK_LO = 312


def _sc_gather_fn():
    mesh = plsc.VectorSubcoreMesh(core_axis_name="c", subcore_axis_name="s")

    @functools.partial(
        pl.kernel,
        mesh=mesh,
        out_type=jax.ShapeDtypeStruct((NPAIR, HEAD_DIM), jnp.float32),
        scratch_types=[
            pltpu.VMEM((CHUNK, NCORNER), jnp.int32),
            pltpu.VMEM((CHUNK, NCORNER), jnp.float32),
            pltpu.VMEM((CHUNK, NCORNER, HEAD_DIM), jnp.float32),
            pltpu.VMEM((CHUNK, HEAD_DIM), jnp.float32),
            pltpu.SemaphoreType.DMA,
        ],
        compiler_params=pltpu.CompilerParams(use_tc_tiling_on_sc=False),
    )
    def sc_gather(vt_hbm, idx_hbm, wgt_hbm, out_hbm, idx_v, wgt_v, rows_v, out_v, sem):
        wid = lax.axis_index("s") * 2 + lax.axis_index("c")
        hi = wid < 16
        base0 = jnp.where(hi, wid * (CHUNK * NCHUNK_HI),
                          16 * CHUNK * NCHUNK_HI + (wid - 16) * (CHUNK * NCHUNK_LO))
        nchunk = jnp.where(hi, NCHUNK_HI, NCHUNK_LO)

        def chunk_body(ci, carry):
            base = base0 + ci * CHUNK
            pltpu.sync_copy(idx_hbm.at[pl.ds(base, CHUNK)], idx_v)
            pltpu.sync_copy(wgt_hbm.at[pl.ds(base, CHUNK)], wgt_v)
            copies = [
                pltpu.async_copy(vt_hbm.at[idx_v.at[j]], rows_v.at[j], sem)
                for j in range(CHUNK)
            ]
            for c in copies:
                c.wait()
            for j in range(CHUNK):
                def g_body(g, accs):
                    a0, a1 = accs
                    w16 = wgt_v[j, pl.ds(g * 16, 16)]
                    for i in range(16):
                        k = g * 16 + i
                        w = w16[i]
                        a0 = a0 + w * rows_v[j, k, pl.ds(0, 16)]
                        a1 = a1 + w * rows_v[j, k, pl.ds(16, 16)]
                    return (a0, a1)
                a0, a1 = lax.fori_loop(
                    0, NCORNER // 16, g_body,
                    (jnp.zeros((16,), jnp.float32), jnp.zeros((16,), jnp.float32)))
                out_v[j, pl.ds(0, 16)] = a0
                out_v[j, pl.ds(16, 16)] = a1
            pltpu.sync_copy(out_v, out_hbm.at[pl.ds(base, CHUNK)])
            return carry

        lax.fori_loop(0, nchunk, chunk_body, 0)

    return sc_gather


def kernel(query, value, reference_points, spatial_shapes, level_start_index,
           W_so, b_so, W_aw, b_aw, W_v, b_v, W_o, b_o):
    q2 = query[0]
    v2 = value[0]
    rp = reference_points[0]
    refx = rp[:, :, 0]
    refy = rp[:, :, 1]

    vt8 = pl.pallas_call(
        _vproj_body,
        grid=(NV // BV,),
        in_specs=[
            pl.BlockSpec((BV, EMBED), lambda i: (i, 0)),
            pl.BlockSpec((EMBED, EMBED), lambda i: (0, 0)),
            pl.BlockSpec((1, EMBED), lambda i: (0, 0)),
        ],
        out_specs=pl.BlockSpec((HEADS, BV, HEAD_DIM), lambda i: (0, i, 0)),
        out_shape=jax.ShapeDtypeStruct((HEADS, NV, HEAD_DIM), jnp.float32),
    )(v2, W_v.T, b_v[None])
    vt = vt8.reshape(HEADS * NV, HEAD_DIM)

    wsoT = W_so.T[:, _PERM]
    bso = b_so[_PERM][None]
    idx8, wgt8 = pl.pallas_call(
        _locs_body,
        grid=(NQ // BQ,),
        in_specs=[
            pl.BlockSpec((BQ, EMBED), lambda i: (i, 0)),
            pl.BlockSpec((BQ, LEVELS), lambda i: (i, 0)),
            pl.BlockSpec((BQ, LEVELS), lambda i: (i, 0)),
            pl.BlockSpec((EMBED, 2 * EMBED), lambda i: (0, 0)),
            pl.BlockSpec((1, 2 * EMBED), lambda i: (0, 0)),
            pl.BlockSpec((EMBED, EMBED), lambda i: (0, 0)),
            pl.BlockSpec((1, EMBED), lambda i: (0, 0)),
            pl.BlockSpec((LEVELS, EMBED), lambda i: (0, 0)),
            pl.BlockSpec((1, EMBED), lambda i: (0, 0)),
            pl.BlockSpec((1, EMBED), lambda i: (0, 0)),
            pl.BlockSpec((1, EMBED), lambda i: (0, 0)),
        ],
        out_specs=[
            pl.BlockSpec((HEADS, BQ, NCORNER), lambda i: (0, i, 0)),
            pl.BlockSpec((HEADS, BQ, NCORNER), lambda i: (0, i, 0)),
        ],
        out_shape=[
            jax.ShapeDtypeStruct((HEADS, NQ, NCORNER), jnp.int32),
            jax.ShapeDtypeStruct((HEADS, NQ, NCORNER), jnp.float32),
        ],
    )(q2, refx, refy, wsoT, bso, W_aw.T, b_aw[None],
      jnp.asarray(_S_LVL), jnp.asarray(_WC_F), jnp.asarray(_WC_I),
      jnp.asarray(_HOFF_C + _START_C))
    idx2 = idx8.reshape(NPAIR, NCORNER)
    wgt2 = wgt8.reshape(NPAIR, NCORNER)

    sc_out = _sc_gather_fn()(vt, idx2, wgt2)
    sc3 = sc_out.reshape(HEADS, NQ, HEAD_DIM)

    out = pl.pallas_call(
        _out_body,
        grid=(NQ // BQ,),
        in_specs=[
            pl.BlockSpec((HEADS, BQ, HEAD_DIM), lambda i: (0, i, 0)),
            pl.BlockSpec((BQ, EMBED), lambda i: (i, 0)),
            pl.BlockSpec((EMBED, EMBED), lambda i: (0, 0)),
            pl.BlockSpec((1, EMBED), lambda i: (0, 0)),
        ],
        out_specs=pl.BlockSpec((BQ, EMBED), lambda i: (i, 0)),
        out_shape=jax.ShapeDtypeStruct((NQ, EMBED), jnp.float32),
    )(sc3, q2, W_o.T, b_o[None])
    return out[None]


# speedup vs baseline: 89.7815x; 1.1170x over previous
import functools

import numpy as np

import jax
import jax.numpy as jnp
from jax import lax
from jax.experimental import pallas as pl
from jax.experimental.pallas import tpu as pltpu
from jax.experimental.pallas import tpu_sc as plsc

EMBED = 256
HEADS = 8
LEVELS = 4
POINTS = 8
HEAD_DIM = 32
LP = LEVELS * POINTS
NQ = 10000
NV = 21760
NPAIR = NQ * HEADS
NCORNER = LP * 4
_LVL_W = np.array([128, 64, 32, 16], dtype=np.int32)
_STARTS = np.array([0, 16384, 20480, 21504], dtype=np.int32)

_ch = np.arange(EMBED)
_l_of = (_ch // POINTS) % LEVELS
_WC_I = np.asarray(_LVL_W[_l_of], np.int32)[None]
_WC_F = _WC_I.astype(np.float32)
_START_C = np.asarray(_STARTS[_l_of], np.int32)[None]
_HOFF_C = np.asarray((_ch // LP) * NV, np.int32)[None]
_S_LVL = np.zeros((LEVELS, EMBED), np.float32)
_S_LVL[_l_of, _ch] = 1.0
_PERM = np.concatenate([2 * np.arange(EMBED), 2 * np.arange(EMBED) + 1])

BQ = 1000
BV = 680


def _vproj_body(v_ref, wvT_ref, bv_ref, out_ref):
    res = jnp.dot(v_ref[...], wvT_ref[...], preferred_element_type=jnp.float32, precision=lax.Precision.HIGHEST)
    res = res + bv_ref[...]
    for h in range(HEADS):
        out_ref[h] = res[:, h * HEAD_DIM:(h + 1) * HEAD_DIM]


def _locs_body(q_ref, refx_ref, refy_ref, wsoT_ref, bso_ref, wawT_ref, baw_ref,
               slvl_ref, wcf_ref, wci_ref, base_ref, idx_ref, wgt_ref):
    wc_f = wcf_ref[...]
    wc_i = wci_ref[...]
    base_c = base_ref[...]

    q = q_ref[...]
    so = jnp.dot(q, wsoT_ref[...], preferred_element_type=jnp.float32, precision=lax.Precision.HIGHEST) + bso_ref[...]
    aw = jnp.dot(q, wawT_ref[...], preferred_element_type=jnp.float32, precision=lax.Precision.HIGHEST) + baw_ref[...]
    aw3 = aw.reshape(BQ, HEADS, LP)
    aw3 = aw3 - jnp.max(aw3, axis=-1, keepdims=True)
    e = jnp.exp(aw3)
    aw = (e / jnp.sum(e, axis=-1, keepdims=True)).reshape(BQ, EMBED)

    rx = jnp.dot(refx_ref[...], slvl_ref[...], preferred_element_type=jnp.float32, precision=lax.Precision.HIGHEST)
    ry = jnp.dot(refy_ref[...], slvl_ref[...], preferred_element_type=jnp.float32, precision=lax.Precision.HIGHEST)
    x = rx * wc_f + so[:, :EMBED] - 0.5
    y = ry * wc_f + so[:, EMBED:] - 0.5
    x = jnp.clip(x, -2.0, wc_f + 1.0)
    y = jnp.clip(y, -2.0, wc_f + 1.0)
    x0 = jnp.floor(x)
    y0 = jnp.floor(y)
    fx = x - x0
    fy = y - y0
    ix = x0.astype(jnp.int32)
    iy = y0.astype(jnp.int32)

    idxs = []
    ws = []
    for dy in (0, 1):
        for dx in (0, 1):
            xi = ix + dx
            yi = iy + dy
            valid = (xi >= 0) & (xi < wc_i) & (yi >= 0) & (yi < wc_i)
            wx = fx if dx else (1.0 - fx)
            wy = fy if dy else (1.0 - fy)
            w = wx * wy * aw * valid.astype(jnp.float32)
            xc = jnp.clip(xi, 0, wc_i - 1)
            yc = jnp.clip(yi, 0, wc_i - 1)
            idxs.append(base_c + yc * wc_i + xc)
            ws.append(w)
    for h in range(HEADS):
        sl = slice(h * LP, (h + 1) * LP)
        idx_ref[h] = jnp.concatenate([c[:, sl] for c in idxs], axis=-1)
        wgt_ref[h] = jnp.concatenate([c[:, sl] for c in ws], axis=-1)


def _out_body(sc_ref, q_ref, woT_ref, bo_ref, out_ref):
    sc = jnp.concatenate([sc_ref[h] for h in range(HEADS)], axis=-1)
    res = jnp.dot(sc, woT_ref[...], preferred_element_type=jnp.float32, precision=lax.Precision.HIGHEST)
    out_ref[...] = res + bo_ref[...] + q_ref[...]


NW = 32
CHUNK = 8
NCHUNK_HI = 314
NCHUNK_LO = 312


def _sc_gather_fn():
    mesh = plsc.VectorSubcoreMesh(core_axis_name="c", subcore_axis_name="s")

    @functools.partial(
        pl.kernel,
        mesh=mesh,
        out_type=jax.ShapeDtypeStruct((NPAIR, HEAD_DIM), jnp.float32),
        scratch_types=[
            pltpu.VMEM((2, CHUNK, NCORNER), jnp.int32),
            pltpu.VMEM((2, CHUNK, NCORNER), jnp.float32),
            pltpu.VMEM((2, CHUNK, NCORNER, HEAD_DIM), jnp.float32),
            pltpu.VMEM((2, CHUNK, HEAD_DIM), jnp.float32),
            pltpu.SemaphoreType.DMA,
            pltpu.SemaphoreType.DMA,
            pltpu.SemaphoreType.DMA,
            pltpu.SemaphoreType.DMA,
            pltpu.SemaphoreType.DMA,
            pltpu.SemaphoreType.DMA,
        ],
        compiler_params=pltpu.CompilerParams(use_tc_tiling_on_sc=False),
    )
    def sc_gather(vt_hbm, idx_hbm, wgt_hbm, out_hbm, idx_v, wgt_v, rows_v, out_v,
                  sem_io0, sem_io1, sem_g0, sem_g1, sem_o0, sem_o1):
        sem_io = (sem_io0, sem_io1)
        sem_g = (sem_g0, sem_g1)
        sem_o = (sem_o0, sem_o1)
        wid = lax.axis_index("s") * 2 + lax.axis_index("c")
        hi = wid < 8
        base0 = jnp.where(hi, wid * (CHUNK * NCHUNK_HI),
                          8 * CHUNK * NCHUNK_HI + (wid - 8) * (CHUNK * NCHUNK_LO))
        nchunk = jnp.where(hi, NCHUNK_HI, NCHUNK_LO)

        def io_src(c):
            return pl.ds(base0 + c * CHUNK, CHUNK)

        def prefetch_io(c, b):
            pltpu.async_copy(idx_hbm.at[io_src(c)], idx_v.at[b], sem_io[b])
            pltpu.async_copy(wgt_hbm.at[io_src(c)], wgt_v.at[b], sem_io[b])

        def wait_io(b):
            pltpu.make_async_copy(idx_hbm.at[pl.ds(0, CHUNK)], idx_v.at[b],
                                  sem_io[b]).wait()
            pltpu.make_async_copy(wgt_hbm.at[pl.ds(0, CHUNK)], wgt_v.at[b],
                                  sem_io[b]).wait()

        def fire_gathers(b):
            for j in range(CHUNK):
                pltpu.async_copy(vt_hbm.at[idx_v.at[b, j]], rows_v.at[b, j],
                                 sem_g[b])

        def wait_gathers(b):
            for j in range(CHUNK):
                pltpu.make_async_copy(vt_hbm.at[idx_v.at[b, j]],
                                      rows_v.at[b, j], sem_g[b]).wait()

        def step(c, b):
            @pl.when(c + 1 < nchunk)
            def _():
                wait_io(1 - b)
                fire_gathers(1 - b)

            @pl.when(c + 2 < nchunk)
            def _():
                prefetch_io(c + 2, b)

            wait_gathers(b)

            @pl.when(c >= 2)
            def _():
                pltpu.make_async_copy(out_v.at[b], out_hbm.at[io_src(0)],
                                      sem_o[b]).wait()

            for j in range(CHUNK):
                def g_body(g, accs):
                    a0, a1 = accs
                    w16 = wgt_v[b, j, pl.ds(g * 16, 16)]
                    for i in range(16):
                        k = g * 16 + i
                        w = w16[i]
                        a0 = a0 + w * rows_v[b, j, k, pl.ds(0, 16)]
                        a1 = a1 + w * rows_v[b, j, k, pl.ds(16, 16)]
                    return (a0, a1)
                a0, a1 = lax.fori_loop(
                    0, NCORNER // 16, g_body,
                    (jnp.zeros((16,), jnp.float32), jnp.zeros((16,), jnp.float32)))
                out_v[b, j, pl.ds(0, 16)] = a0
                out_v[b, j, pl.ds(16, 16)] = a1
            pltpu.async_copy(out_v.at[b], out_hbm.at[io_src(c)], sem_o[b])

        prefetch_io(0, 0)
        prefetch_io(1, 1)
        wait_io(0)
        fire_gathers(0)

        def pair_body(c2, carry):
            step(2 * c2, 0)
            step(2 * c2 + 1, 1)
            return carry

        lax.fori_loop(0, nchunk // 2, pair_body, 0)

        pltpu.make_async_copy(out_v.at[0], out_hbm.at[io_src(0)], sem_o0).wait()
        pltpu.make_async_copy(out_v.at[1], out_hbm.at[io_src(0)], sem_o1).wait()

    return sc_gather


def kernel(query, value, reference_points, spatial_shapes, level_start_index,
           W_so, b_so, W_aw, b_aw, W_v, b_v, W_o, b_o):
    q2 = query[0]
    v2 = value[0]
    rp = reference_points[0]
    refx = rp[:, :, 0]
    refy = rp[:, :, 1]

    vt8 = pl.pallas_call(
        _vproj_body,
        grid=(NV // BV,),
        in_specs=[
            pl.BlockSpec((BV, EMBED), lambda i: (i, 0)),
            pl.BlockSpec((EMBED, EMBED), lambda i: (0, 0)),
            pl.BlockSpec((1, EMBED), lambda i: (0, 0)),
        ],
        out_specs=pl.BlockSpec((HEADS, BV, HEAD_DIM), lambda i: (0, i, 0)),
        out_shape=jax.ShapeDtypeStruct((HEADS, NV, HEAD_DIM), jnp.float32),
    )(v2, W_v.T, b_v[None])
    vt = vt8.reshape(HEADS * NV, HEAD_DIM)

    wsoT = W_so.T[:, _PERM]
    bso = b_so[_PERM][None]
    idx8, wgt8 = pl.pallas_call(
        _locs_body,
        grid=(NQ // BQ,),
        in_specs=[
            pl.BlockSpec((BQ, EMBED), lambda i: (i, 0)),
            pl.BlockSpec((BQ, LEVELS), lambda i: (i, 0)),
            pl.BlockSpec((BQ, LEVELS), lambda i: (i, 0)),
            pl.BlockSpec((EMBED, 2 * EMBED), lambda i: (0, 0)),
            pl.BlockSpec((1, 2 * EMBED), lambda i: (0, 0)),
            pl.BlockSpec((EMBED, EMBED), lambda i: (0, 0)),
            pl.BlockSpec((1, EMBED), lambda i: (0, 0)),
            pl.BlockSpec((LEVELS, EMBED), lambda i: (0, 0)),
            pl.BlockSpec((1, EMBED), lambda i: (0, 0)),
            pl.BlockSpec((1, EMBED), lambda i: (0, 0)),
            pl.BlockSpec((1, EMBED), lambda i: (0, 0)),
        ],
        out_specs=[
            pl.BlockSpec((HEADS, BQ, NCORNER), lambda i: (0, i, 0)),
            pl.BlockSpec((HEADS, BQ, NCORNER), lambda i: (0, i, 0)),
        ],
        out_shape=[
            jax.ShapeDtypeStruct((HEADS, NQ, NCORNER), jnp.int32),
            jax.ShapeDtypeStruct((HEADS, NQ, NCORNER), jnp.float32),
        ],
    )(q2, refx, refy, wsoT, bso, W_aw.T, b_aw[None],
      jnp.asarray(_S_LVL), jnp.asarray(_WC_F), jnp.asarray(_WC_I),
      jnp.asarray(_HOFF_C + _START_C))
    idx2 = idx8.reshape(NPAIR, NCORNER)
    wgt2 = wgt8.reshape(NPAIR, NCORNER)

    sc_out = _sc_gather_fn()(vt, idx2, wgt2)
    sc3 = sc_out.reshape(HEADS, NQ, HEAD_DIM)

    out = pl.pallas_call(
        _out_body,
        grid=(NQ // BQ,),
        in_specs=[
            pl.BlockSpec((HEADS, BQ, HEAD_DIM), lambda i: (0, i, 0)),
            pl.BlockSpec((BQ, EMBED), lambda i: (i, 0)),
            pl.BlockSpec((EMBED, EMBED), lambda i: (0, 0)),
            pl.BlockSpec((1, EMBED), lambda i: (0, 0)),
        ],
        out_specs=pl.BlockSpec((BQ, EMBED), lambda i: (i, 0)),
        out_shape=jax.ShapeDtypeStruct((NQ, EMBED), jnp.float32),
    )(sc3, q2, W_o.T, b_o[None])
    return out[None]
